# baseline (speedup 1.0000x reference)
import functools

import jax
import jax.numpy as jnp
from jax import lax
from jax.experimental import pallas as pl
from jax.experimental.pallas import tpu as pltpu
from jax.experimental.pallas import tpu_sc as plsc

_N = 10000
_D = 128
_DH = 64
_DQ = 32
_E = 320000

_NC = 2
_NS = 16
_CH = 128
_N_PAD = 10240
_RPT = _N_PAD // _NS
_EP = 327680
_NCHUNKS = _EP // _CH
_CPT = _NCHUNKS // _NS
_CPW = _NCHUNKS // (_NC * _NS)
_DEGW = 16
_K = 5
_KH = 1

_mesh = plsc.VectorSubcoreMesh(core_axis_name="c", subcore_axis_name="s")
_sc_params = pltpu.CompilerParams(use_tc_tiling_on_sc=False)


@functools.partial(
    pl.kernel,
    out_type=[
        jax.ShapeDtypeStruct((_N_PAD, _DEGW), jnp.float32),
        jax.ShapeDtypeStruct((_N_PAD, _DEGW), jnp.float32),
    ],
    mesh=_mesh,
    scratch_types=[
        pltpu.VMEM((_CPW, _CH), jnp.int32),
        pltpu.VMEM((_CH, _DEGW), jnp.float32),
        pltpu.VMEM((_RPT, _DEGW), jnp.float32),
        pltpu.VMEM_SHARED((_N_PAD, _DEGW), jnp.float32),
    ],
    compiler_params=_sc_params,
)
def _deg_call(dst_hbm, deg0_hbm, deg1_hbm, idx_v, ones_v, z_v, deg_sp):
    c = lax.axis_index("c")
    s = lax.axis_index("s")
    w = s * _NC + c
    rows = pl.ds(s * _RPT, _RPT)

    def fill_ones(i, carry):
        ones_v[i, :] = jnp.ones((16,), jnp.float32)
        return carry

    lax.fori_loop(0, _CH, fill_ones, 0)

    def fill_zero(i, carry):
        z_v[i, :] = jnp.zeros((16,), jnp.float32)
        return carry

    lax.fori_loop(0, _RPT, fill_zero, 0)

    pltpu.sync_copy(z_v, deg_sp.at[rows])
    pltpu.sync_copy(dst_hbm.at[pl.ds(w * _CPW, _CPW)], idx_v)
    plsc.subcore_barrier()

    def add_chunk(j, carry):
        pltpu.sync_copy(ones_v, deg_sp.at[idx_v.at[j]], add=True)
        return carry

    lax.fori_loop(0, _CPW, add_chunk, 0)
    plsc.subcore_barrier()

    @pl.when(c == 0)
    def _():
        pltpu.sync_copy(deg_sp.at[rows], deg0_hbm.at[rows])

    @pl.when(c == 1)
    def _():
        pltpu.sync_copy(deg_sp.at[rows], deg1_hbm.at[rows])


@functools.partial(
    pl.kernel,
    out_type=[
        jax.ShapeDtypeStruct((_N_PAD, _DH), jnp.float32),
        jax.ShapeDtypeStruct((_N_PAD, _DH), jnp.float32),
    ],
    mesh=_mesh,
    scratch_types=[
        pltpu.VMEM((_CPT, _CH), jnp.int32),
        pltpu.VMEM((_CPT, _CH), jnp.int32),
        [pltpu.VMEM((_CH, _DQ), jnp.float32)] * _K,
        [pltpu.SemaphoreType.DMA] * _K,
        [pltpu.SemaphoreType.DMA] * _K,
        pltpu.VMEM_SHARED((_N_PAD, _DQ), jnp.float32),
        pltpu.VMEM_SHARED((_N_PAD, _DQ), jnp.float32),
    ],
    compiler_params=_sc_params,
)
def _scatter_call(q00_hbm, q01_hbm, q10_hbm, q11_hbm, src_hbm, dst_hbm,
                  o0_hbm, o1_hbm, idx_s, idx_d, gbufs, gsems, ssems,
                  acc, hs_sp):
    c = lax.axis_index("c")
    s = lax.axis_index("s")
    rows = pl.ds(s * _RPT, _RPT)
    chunks = pl.ds(s * _CPT, _CPT)

    pltpu.sync_copy(src_hbm.at[chunks], idx_s)
    pltpu.sync_copy(dst_hbm.at[chunks], idx_d)

    def run(qs, o_hbm):
        for h in range(2):
            q_hbm = qs[h]
            cols = pl.ds(h * _DQ, _DQ)
            pltpu.sync_copy(q_hbm.at[rows], acc.at[rows])
            pltpu.sync_copy(q_hbm.at[rows], hs_sp.at[rows])
            plsc.subcore_barrier()

            def gsrc(b):
                return q_hbm if b < _KH else hs_sp

            for b in range(_K):
                pltpu.async_copy(gsrc(b).at[idx_s.at[b]], gbufs[b], gsems[b])

            def body(t, carry):
                for b in range(_K):
                    cc = t * _K + b
                    pltpu.make_async_copy(gsrc(b).at[idx_s.at[cc]], gbufs[b],
                                          gsems[b]).wait()
                    pltpu.async_copy(gbufs[b], acc.at[idx_d.at[cc]],
                                     ssems[b], add=True)
                    bp = (b - 1) % _K
                    cp = cc - 1

                    @pl.when(cp >= 0)
                    def _():
                        pltpu.make_async_copy(gbufs[bp],
                                              acc.at[idx_d.at[cp]],
                                              ssems[bp]).wait()

                        @pl.when(cp + _K < _CPT)
                        def _():
                            pltpu.async_copy(gsrc(bp).at[idx_s.at[cp + _K]],
                                             gbufs[bp], gsems[bp])
                return carry

            lax.fori_loop(0, _CPT // _K, body, 0)
            pltpu.make_async_copy(gbufs[(_CPT - 1) % _K],
                                  acc.at[idx_d.at[_CPT - 1]],
                                  ssems[(_CPT - 1) % _K]).wait()
            plsc.subcore_barrier()
            pltpu.sync_copy(acc.at[rows], o_hbm.at[rows, cols])
            plsc.subcore_barrier()

    @pl.when(c == 0)
    def _():
        run((q00_hbm, q01_hbm), o0_hbm)

    @pl.when(c == 1)
    def _():
        run((q10_hbm, q11_hbm), o1_hbm)


_BM = 1024
_GRID = (_N_PAD // _BM,)


def _dis_block(deg0_ref, deg1_ref):
    return lax.rsqrt(deg0_ref[...] + deg1_ref[...] + 1.0)


def _mm1_body(x_ref, w_ref, deg0_ref, deg1_ref, hs0_ref, hs1_ref, hs2_ref, hs3_ref):
    dis = _dis_block(deg0_ref, deg1_ref)
    h = jnp.dot(x_ref[...], w_ref[...], preferred_element_type=jnp.float32)
    hs = h * dis
    hs0_ref[...] = hs[:, 0 * _DQ:1 * _DQ]
    hs1_ref[...] = hs[:, 1 * _DQ:2 * _DQ]
    hs2_ref[...] = hs[:, 2 * _DQ:3 * _DQ]
    hs3_ref[...] = hs[:, 3 * _DQ:4 * _DQ]


def _mm1_call(x, W, deg0, deg1):
    return pl.pallas_call(
        _mm1_body,
        grid=_GRID,
        in_specs=[
            pl.BlockSpec((_BM, _D), lambda i: (i, 0)),
            pl.BlockSpec((_D, _D), lambda i: (0, 0)),
            pl.BlockSpec((_BM, 1), lambda i: (i, 0)),
            pl.BlockSpec((_BM, 1), lambda i: (i, 0)),
        ],
        out_specs=[pl.BlockSpec((_BM, _DQ), lambda i: (i, 0))] * 4,
        out_shape=[jax.ShapeDtypeStruct((_N_PAD, _DQ), jnp.float32)] * 4,
    )(x, W, deg0, deg1)


def _comb_body(o0_ref, o1_ref, deg0_ref, deg1_ref, b_ref, w_ref,
               hs0_ref, hs1_ref, hs2_ref, hs3_ref):
    dis = _dis_block(deg0_ref, deg1_ref)
    t = jnp.concatenate([o0_ref[...], o1_ref[...]], axis=1)
    a = jnp.maximum(t * dis + b_ref[...], 0.0)
    h = jnp.dot(a, w_ref[...], preferred_element_type=jnp.float32)
    hs = h * dis
    hs0_ref[...] = hs[:, 0 * _DQ:1 * _DQ]
    hs1_ref[...] = hs[:, 1 * _DQ:2 * _DQ]
    hs2_ref[...] = hs[:, 2 * _DQ:3 * _DQ]
    hs3_ref[...] = hs[:, 3 * _DQ:4 * _DQ]


def _comb_call(o0, o1, deg0, deg1, b, W):
    return pl.pallas_call(
        _comb_body,
        grid=_GRID,
        in_specs=[
            pl.BlockSpec((_BM, _DH), lambda i: (i, 0)),
            pl.BlockSpec((_BM, _DH), lambda i: (i, 0)),
            pl.BlockSpec((_BM, 1), lambda i: (i, 0)),
            pl.BlockSpec((_BM, 1), lambda i: (i, 0)),
            pl.BlockSpec((1, _D), lambda i: (0, 0)),
            pl.BlockSpec((_D, _D), lambda i: (0, 0)),
        ],
        out_specs=[pl.BlockSpec((_BM, _DQ), lambda i: (i, 0))] * 4,
        out_shape=[jax.ShapeDtypeStruct((_N_PAD, _DQ), jnp.float32)] * 4,
    )(o0, o1, deg0, deg1, b, W)


def _final_body(o0_ref, o1_ref, deg0_ref, deg1_ref, b_ref, out_ref):
    dis = _dis_block(deg0_ref, deg1_ref)
    t = jnp.concatenate([o0_ref[...], o1_ref[...]], axis=1)
    out_ref[...] = t * dis + b_ref[...]


def _final_call(o0, o1, deg0, deg1, b):
    return pl.pallas_call(
        _final_body,
        grid=_GRID,
        in_specs=[
            pl.BlockSpec((_BM, _DH), lambda i: (i, 0)),
            pl.BlockSpec((_BM, _DH), lambda i: (i, 0)),
            pl.BlockSpec((_BM, 1), lambda i: (i, 0)),
            pl.BlockSpec((_BM, 1), lambda i: (i, 0)),
            pl.BlockSpec((1, _D), lambda i: (0, 0)),
        ],
        out_specs=pl.BlockSpec((_BM, _D), lambda i: (i, 0)),
        out_shape=jax.ShapeDtypeStruct((_N_PAD, _D), jnp.float32),
    )(o0, o1, deg0, deg1, b)


def kernel(x, edge_index, W1, b1, W2, b2, W3, b3):
    src = edge_index[0].astype(jnp.int32)
    dst = edge_index[1].astype(jnp.int32)
    pad = _EP - _E
    fill = jnp.full((pad,), _N, jnp.int32)
    src_p = jnp.concatenate([src, fill]).reshape(_NCHUNKS, _CH)
    dst_p = jnp.concatenate([dst, fill]).reshape(_NCHUNKS, _CH)
    x_p = jnp.pad(x.astype(jnp.float32), ((0, _N_PAD - _N), (0, 0)))
    b1r = b1.reshape(1, _D)
    b2r = b2.reshape(1, _D)
    b3r = b3.reshape(1, _D)

    deg0, deg1 = _deg_call(dst_p)
    deg0 = deg0[:, :1]
    deg1 = deg1[:, :1]
    q = _mm1_call(x_p, W1, deg0, deg1)
    o0, o1 = _scatter_call(*q, src_p, dst_p)
    q = _comb_call(o0, o1, deg0, deg1, b1r, W2)
    o0, o1 = _scatter_call(*q, src_p, dst_p)
    q = _comb_call(o0, o1, deg0, deg1, b2r, W3)
    o0, o1 = _scatter_call(*q, src_p, dst_p)
    out = _final_call(o0, o1, deg0, deg1, b3r)
    return out[:_N]


# speedup vs baseline: 1.1701x; 1.1701x over previous
import functools

import jax
import jax.numpy as jnp
from jax import lax
from jax.experimental import pallas as pl
from jax.experimental.pallas import tpu as pltpu
from jax.experimental.pallas import tpu_sc as plsc

_N = 10000
_D = 128
_DH = 64
_DQ = 32
_E = 320000

_NC = 2
_NS = 16
_CH = 128
_N_PAD = 10240
_RPT = _N_PAD // _NS
_EP = 327680
_NCHUNKS = _EP // _CH
_CPT = _NCHUNKS // _NS
_CPW = _NCHUNKS // (_NC * _NS)
_DEGW = 16
_K = 5

_mesh = plsc.VectorSubcoreMesh(core_axis_name="c", subcore_axis_name="s")
_sc_params = pltpu.CompilerParams(use_tc_tiling_on_sc=False)


@functools.partial(
    pl.kernel,
    out_type=[
        jax.ShapeDtypeStruct((_N_PAD, _DEGW), jnp.float32),
        jax.ShapeDtypeStruct((_N_PAD, _DEGW), jnp.float32),
    ],
    mesh=_mesh,
    scratch_types=[
        pltpu.VMEM((_CPW, _CH), jnp.int32),
        pltpu.VMEM((_CH, _DEGW), jnp.float32),
        pltpu.VMEM((_RPT, _DEGW), jnp.float32),
        pltpu.VMEM_SHARED((_N_PAD, _DEGW), jnp.float32),
    ],
    compiler_params=_sc_params,
)
def _deg_call(dst_hbm, deg0_hbm, deg1_hbm, idx_v, ones_v, z_v, deg_sp):
    c = lax.axis_index("c")
    s = lax.axis_index("s")
    w = s * _NC + c
    rows = pl.ds(s * _RPT, _RPT)

    def fill_ones(i, carry):
        ones_v[i, :] = jnp.ones((16,), jnp.float32)
        return carry

    lax.fori_loop(0, _CH, fill_ones, 0)

    def fill_zero(i, carry):
        z_v[i, :] = jnp.zeros((16,), jnp.float32)
        return carry

    lax.fori_loop(0, _RPT, fill_zero, 0)

    pltpu.sync_copy(z_v, deg_sp.at[rows])
    pltpu.sync_copy(dst_hbm.at[pl.ds(w * _CPW, _CPW)], idx_v)
    plsc.subcore_barrier()

    def add_chunk(j, carry):
        pltpu.sync_copy(ones_v, deg_sp.at[idx_v.at[j]], add=True)
        return carry

    lax.fori_loop(0, _CPW, add_chunk, 0)
    plsc.subcore_barrier()

    @pl.when(c == 0)
    def _():
        pltpu.sync_copy(deg_sp.at[rows], deg0_hbm.at[rows])

    @pl.when(c == 1)
    def _():
        pltpu.sync_copy(deg_sp.at[rows], deg1_hbm.at[rows])


@functools.partial(
    pl.kernel,
    out_type=[
        jax.ShapeDtypeStruct((_N_PAD, _DH), jnp.float32),
        jax.ShapeDtypeStruct((_N_PAD, _DH), jnp.float32),
    ],
    mesh=_mesh,
    scratch_types=[
        pltpu.VMEM((_CPT, _CH), jnp.int32),
        pltpu.VMEM((_CPT, _CH), jnp.int32),
        [pltpu.VMEM((_CH, _DQ), jnp.float32)] * _K,
        [pltpu.SemaphoreType.DMA] * _K,
        [pltpu.SemaphoreType.DMA] * _K,
        pltpu.VMEM_SHARED((_N_PAD, _DQ), jnp.float32),
        pltpu.VMEM_SHARED((_N_PAD, _DQ), jnp.float32),
    ],
    compiler_params=_sc_params,
)
def _scatter_call(hs0_hbm, hs1_hbm, src_hbm, dst_hbm, o0_hbm, o1_hbm,
                  idx_s, idx_d, gbufs, gsems, ssems, acc, hs_sp):
    c = lax.axis_index("c")
    s = lax.axis_index("s")
    rows = pl.ds(s * _RPT, _RPT)
    chunks = pl.ds(s * _CPT, _CPT)

    pltpu.sync_copy(src_hbm.at[chunks], idx_s)
    pltpu.sync_copy(dst_hbm.at[chunks], idx_d)

    def run(hs_hbm, o_hbm):
        for h in range(2):
            cols = pl.ds(h * _DQ, _DQ)
            pltpu.sync_copy(hs_hbm.at[rows, cols], acc.at[rows])
            pltpu.sync_copy(hs_hbm.at[rows, cols], hs_sp.at[rows])
            plsc.subcore_barrier()

            for b in range(_K):
                pltpu.async_copy(hs_sp.at[idx_s.at[b]], gbufs[b], gsems[b])

            def body(t, carry):
                for b in range(_K):
                    cc = t * _K + b
                    pltpu.make_async_copy(hs_sp.at[idx_s.at[cc]], gbufs[b],
                                          gsems[b]).wait()
                    pltpu.async_copy(gbufs[b], acc.at[idx_d.at[cc]],
                                     ssems[b], add=True)
                    bp = (b - 1) % _K
                    cp = cc - 1

                    @pl.when(cp >= 0)
                    def _():
                        pltpu.make_async_copy(gbufs[bp],
                                              acc.at[idx_d.at[cp]],
                                              ssems[bp]).wait()

                        @pl.when(cp + _K < _CPT)
                        def _():
                            pltpu.async_copy(hs_sp.at[idx_s.at[cp + _K]],
                                             gbufs[bp], gsems[bp])
                return carry

            lax.fori_loop(0, _CPT // _K, body, 0)
            pltpu.make_async_copy(gbufs[(_CPT - 1) % _K],
                                  acc.at[idx_d.at[_CPT - 1]],
                                  ssems[(_CPT - 1) % _K]).wait()
            plsc.subcore_barrier()
            pltpu.sync_copy(acc.at[rows], o_hbm.at[rows, cols])
            plsc.subcore_barrier()

    @pl.when(c == 0)
    def _():
        run(hs0_hbm, o0_hbm)

    @pl.when(c == 1)
    def _():
        run(hs1_hbm, o1_hbm)


_BM = 1024
_GRID = (_N_PAD // _BM,)


def _dis_block(deg0_ref, deg1_ref):
    return lax.rsqrt(deg0_ref[...] + deg1_ref[...] + 1.0)


def _mm1_body(x_ref, w_ref, deg0_ref, deg1_ref, hs0_ref, hs1_ref):
    dis = _dis_block(deg0_ref, deg1_ref)
    h = jnp.dot(x_ref[...], w_ref[...], preferred_element_type=jnp.float32)
    hs = h * dis
    hs0_ref[...] = hs[:, :_DH]
    hs1_ref[...] = hs[:, _DH:]


def _mm1_call(x, W, deg0, deg1):
    return pl.pallas_call(
        _mm1_body,
        grid=_GRID,
        in_specs=[
            pl.BlockSpec((_BM, _D), lambda i: (i, 0)),
            pl.BlockSpec((_D, _D), lambda i: (0, 0)),
            pl.BlockSpec((_BM, 1), lambda i: (i, 0)),
            pl.BlockSpec((_BM, 1), lambda i: (i, 0)),
        ],
        out_specs=[
            pl.BlockSpec((_BM, _DH), lambda i: (i, 0)),
            pl.BlockSpec((_BM, _DH), lambda i: (i, 0)),
        ],
        out_shape=[
            jax.ShapeDtypeStruct((_N_PAD, _DH), jnp.float32),
            jax.ShapeDtypeStruct((_N_PAD, _DH), jnp.float32),
        ],
    )(x, W, deg0, deg1)


def _comb_body(o0_ref, o1_ref, deg0_ref, deg1_ref, b_ref, w_ref,
               hs0_ref, hs1_ref):
    dis = _dis_block(deg0_ref, deg1_ref)
    t = jnp.concatenate([o0_ref[...], o1_ref[...]], axis=1)
    a = jnp.maximum(t * dis + b_ref[...], 0.0)
    h = jnp.dot(a, w_ref[...], preferred_element_type=jnp.float32)
    hs = h * dis
    hs0_ref[...] = hs[:, :_DH]
    hs1_ref[...] = hs[:, _DH:]


def _comb_call(o0, o1, deg0, deg1, b, W):
    return pl.pallas_call(
        _comb_body,
        grid=_GRID,
        in_specs=[
            pl.BlockSpec((_BM, _DH), lambda i: (i, 0)),
            pl.BlockSpec((_BM, _DH), lambda i: (i, 0)),
            pl.BlockSpec((_BM, 1), lambda i: (i, 0)),
            pl.BlockSpec((_BM, 1), lambda i: (i, 0)),
            pl.BlockSpec((1, _D), lambda i: (0, 0)),
            pl.BlockSpec((_D, _D), lambda i: (0, 0)),
        ],
        out_specs=[
            pl.BlockSpec((_BM, _DH), lambda i: (i, 0)),
            pl.BlockSpec((_BM, _DH), lambda i: (i, 0)),
        ],
        out_shape=[
            jax.ShapeDtypeStruct((_N_PAD, _DH), jnp.float32),
            jax.ShapeDtypeStruct((_N_PAD, _DH), jnp.float32),
        ],
    )(o0, o1, deg0, deg1, b, W)


def _final_body(o0_ref, o1_ref, deg0_ref, deg1_ref, b_ref, out_ref):
    dis = _dis_block(deg0_ref, deg1_ref)
    t = jnp.concatenate([o0_ref[...], o1_ref[...]], axis=1)
    out_ref[...] = t * dis + b_ref[...]


def _final_call(o0, o1, deg0, deg1, b):
    return pl.pallas_call(
        _final_body,
        grid=_GRID,
        in_specs=[
            pl.BlockSpec((_BM, _DH), lambda i: (i, 0)),
            pl.BlockSpec((_BM, _DH), lambda i: (i, 0)),
            pl.BlockSpec((_BM, 1), lambda i: (i, 0)),
            pl.BlockSpec((_BM, 1), lambda i: (i, 0)),
            pl.BlockSpec((1, _D), lambda i: (0, 0)),
        ],
        out_specs=pl.BlockSpec((_BM, _D), lambda i: (i, 0)),
        out_shape=jax.ShapeDtypeStruct((_N_PAD, _D), jnp.float32),
    )(o0, o1, deg0, deg1, b)


def kernel(x, edge_index, W1, b1, W2, b2, W3, b3):
    src = edge_index[0].astype(jnp.int32)
    dst = edge_index[1].astype(jnp.int32)
    pad = _EP - _E
    fill = jnp.full((pad,), _N, jnp.int32)
    src_p = jnp.concatenate([src, fill]).reshape(_NCHUNKS, _CH)
    dst_p = jnp.concatenate([dst, fill]).reshape(_NCHUNKS, _CH)
    x_p = jnp.pad(x.astype(jnp.float32), ((0, _N_PAD - _N), (0, 0)))
    b1r = b1.reshape(1, _D)
    b2r = b2.reshape(1, _D)
    b3r = b3.reshape(1, _D)

    deg0, deg1 = _deg_call(dst_p)
    deg0 = deg0[:, :1]
    deg1 = deg1[:, :1]
    hs0, hs1 = _mm1_call(x_p, W1, deg0, deg1)
    o0, o1 = _scatter_call(hs0, hs1, src_p, dst_p)
    hs0, hs1 = _comb_call(o0, o1, deg0, deg1, b1r, W2)
    o0, o1 = _scatter_call(hs0, hs1, src_p, dst_p)
    hs0, hs1 = _comb_call(o0, o1, deg0, deg1, b2r, W3)
    o0, o1 = _scatter_call(hs0, hs1, src_p, dst_p)
    out = _final_call(o0, o1, deg0, deg1, b3r)
    return out[:_N]


# speedup vs baseline: 1.2010x; 1.0264x over previous
import functools

import jax
import jax.numpy as jnp
from jax import lax
from jax.experimental import pallas as pl
from jax.experimental.pallas import tpu as pltpu
from jax.experimental.pallas import tpu_sc as plsc

_N = 10000
_D = 128
_DH = 64
_DQ = 32
_E = 320000

_NC = 2
_NS = 16
_CH = 128
_N_PAD = 10240
_RPT = _N_PAD // _NS
_EP = 327680
_NCHUNKS = _EP // _CH
_CPT = _NCHUNKS // _NS
_CPW = _NCHUNKS // (_NC * _NS)
_DEGW = 16
_K = 8

_mesh = plsc.VectorSubcoreMesh(core_axis_name="c", subcore_axis_name="s")
_sc_params = pltpu.CompilerParams(use_tc_tiling_on_sc=False)


@functools.partial(
    pl.kernel,
    out_type=[
        jax.ShapeDtypeStruct((_N_PAD, _DEGW), jnp.float32),
        jax.ShapeDtypeStruct((_N_PAD, _DEGW), jnp.float32),
    ],
    mesh=_mesh,
    scratch_types=[
        pltpu.VMEM((_CPW, _CH), jnp.int32),
        pltpu.VMEM((_CH, _DEGW), jnp.float32),
        pltpu.VMEM((_RPT, _DEGW), jnp.float32),
        pltpu.VMEM_SHARED((_N_PAD, _DEGW), jnp.float32),
    ],
    compiler_params=_sc_params,
)
def _deg_call(dst_hbm, deg0_hbm, deg1_hbm, idx_v, ones_v, z_v, deg_sp):
    c = lax.axis_index("c")
    s = lax.axis_index("s")
    w = s * _NC + c
    rows = pl.ds(s * _RPT, _RPT)

    def fill_ones(i, carry):
        ones_v[i, :] = jnp.ones((16,), jnp.float32)
        return carry

    lax.fori_loop(0, _CH, fill_ones, 0)

    def fill_zero(i, carry):
        z_v[i, :] = jnp.zeros((16,), jnp.float32)
        return carry

    lax.fori_loop(0, _RPT, fill_zero, 0)

    pltpu.sync_copy(z_v, deg_sp.at[rows])
    pltpu.sync_copy(dst_hbm.at[pl.ds(w * _CPW, _CPW)], idx_v)
    plsc.subcore_barrier()

    def add_chunk(j, carry):
        pltpu.sync_copy(ones_v, deg_sp.at[idx_v.at[j]], add=True)
        return carry

    lax.fori_loop(0, _CPW, add_chunk, 0)
    plsc.subcore_barrier()

    @pl.when(c == 0)
    def _():
        pltpu.sync_copy(deg_sp.at[rows], deg0_hbm.at[rows])

    @pl.when(c == 1)
    def _():
        pltpu.sync_copy(deg_sp.at[rows], deg1_hbm.at[rows])


@functools.partial(
    pl.kernel,
    out_type=[
        jax.ShapeDtypeStruct((_N_PAD, _DH), jnp.float32),
        jax.ShapeDtypeStruct((_N_PAD, _DH), jnp.float32),
    ],
    mesh=_mesh,
    scratch_types=[
        pltpu.VMEM((_CPT, _CH), jnp.int32),
        pltpu.VMEM((_CPT, _CH), jnp.int32),
        [pltpu.VMEM((_CH, _DQ), jnp.float32)] * _K,
        [pltpu.SemaphoreType.DMA] * _K,
        [pltpu.SemaphoreType.DMA] * _K,
        pltpu.VMEM_SHARED((_N_PAD, _DQ), jnp.float32),
        pltpu.VMEM_SHARED((_N_PAD, _DQ), jnp.float32),
    ],
    compiler_params=_sc_params,
)
def _scatter_call(hs0_hbm, hs1_hbm, src_hbm, dst_hbm, o0_hbm, o1_hbm,
                  idx_s, idx_d, gbufs, gsems, ssems, acc, hs_sp):
    c = lax.axis_index("c")
    s = lax.axis_index("s")
    rows = pl.ds(s * _RPT, _RPT)
    chunks = pl.ds(s * _CPT, _CPT)


    isem = gsems[0]
    dsem = ssems[0]
    icp = pltpu.async_copy(src_hbm.at[chunks], idx_s, isem)
    dcp = pltpu.async_copy(dst_hbm.at[chunks], idx_d, dsem)

    def run(hs_hbm, o_hbm):
        for h in range(2):
            cols = pl.ds(h * _DQ, _DQ)
            pltpu.sync_copy(hs_hbm.at[rows, cols], acc.at[rows])
            pltpu.sync_copy(hs_hbm.at[rows, cols], hs_sp.at[rows])
            if h == 0:
                icp.wait()
                dcp.wait()
            plsc.subcore_barrier()

            for b in range(_K):
                pltpu.async_copy(hs_sp.at[idx_s.at[b]], gbufs[b], gsems[b])

            def body(t, carry):
                for b in range(_K):
                    cc = t * _K + b
                    pltpu.make_async_copy(hs_sp.at[idx_s.at[cc]], gbufs[b],
                                          gsems[b]).wait()
                    pltpu.async_copy(gbufs[b], acc.at[idx_d.at[cc]],
                                     ssems[b], add=True)
                    bp = (b - 1) % _K
                    cp = cc - 1

                    @pl.when(cp >= 0)
                    def _():
                        pltpu.make_async_copy(gbufs[bp],
                                              acc.at[idx_d.at[cp]],
                                              ssems[bp]).wait()

                        @pl.when(cp + _K < _CPT)
                        def _():
                            pltpu.async_copy(hs_sp.at[idx_s.at[cp + _K]],
                                             gbufs[bp], gsems[bp])
                return carry

            lax.fori_loop(0, _CPT // _K, body, 0)
            pltpu.make_async_copy(gbufs[(_CPT - 1) % _K],
                                  acc.at[idx_d.at[_CPT - 1]],
                                  ssems[(_CPT - 1) % _K]).wait()
            plsc.subcore_barrier()
            pltpu.sync_copy(acc.at[rows], o_hbm.at[rows, cols])

    @pl.when(c == 0)
    def _():
        run(hs0_hbm, o0_hbm)

    @pl.when(c == 1)
    def _():
        run(hs1_hbm, o1_hbm)


_BM = 1024
_GRID = (_N_PAD // _BM,)


def _dis_block(deg0_ref, deg1_ref):
    return lax.rsqrt(deg0_ref[...] + deg1_ref[...] + 1.0)


def _mm1_body(x_ref, w_ref, deg0_ref, deg1_ref, hs0_ref, hs1_ref):
    dis = _dis_block(deg0_ref, deg1_ref)
    h = jnp.dot(x_ref[...], w_ref[...], preferred_element_type=jnp.float32)
    hs = h * dis
    hs0_ref[...] = hs[:, :_DH]
    hs1_ref[...] = hs[:, _DH:]


def _mm1_call(x, W, deg0, deg1):
    return pl.pallas_call(
        _mm1_body,
        grid=_GRID,
        in_specs=[
            pl.BlockSpec((_BM, _D), lambda i: (i, 0)),
            pl.BlockSpec((_D, _D), lambda i: (0, 0)),
            pl.BlockSpec((_BM, 1), lambda i: (i, 0)),
            pl.BlockSpec((_BM, 1), lambda i: (i, 0)),
        ],
        out_specs=[
            pl.BlockSpec((_BM, _DH), lambda i: (i, 0)),
            pl.BlockSpec((_BM, _DH), lambda i: (i, 0)),
        ],
        out_shape=[
            jax.ShapeDtypeStruct((_N_PAD, _DH), jnp.float32),
            jax.ShapeDtypeStruct((_N_PAD, _DH), jnp.float32),
        ],
    )(x, W, deg0, deg1)


def _comb_body(o0_ref, o1_ref, deg0_ref, deg1_ref, b_ref, w_ref,
               hs0_ref, hs1_ref):
    dis = _dis_block(deg0_ref, deg1_ref)
    t = jnp.concatenate([o0_ref[...], o1_ref[...]], axis=1)
    a = jnp.maximum(t * dis + b_ref[...], 0.0)
    h = jnp.dot(a, w_ref[...], preferred_element_type=jnp.float32)
    hs = h * dis
    hs0_ref[...] = hs[:, :_DH]
    hs1_ref[...] = hs[:, _DH:]


def _comb_call(o0, o1, deg0, deg1, b, W):
    return pl.pallas_call(
        _comb_body,
        grid=_GRID,
        in_specs=[
            pl.BlockSpec((_BM, _DH), lambda i: (i, 0)),
            pl.BlockSpec((_BM, _DH), lambda i: (i, 0)),
            pl.BlockSpec((_BM, 1), lambda i: (i, 0)),
            pl.BlockSpec((_BM, 1), lambda i: (i, 0)),
            pl.BlockSpec((1, _D), lambda i: (0, 0)),
            pl.BlockSpec((_D, _D), lambda i: (0, 0)),
        ],
        out_specs=[
            pl.BlockSpec((_BM, _DH), lambda i: (i, 0)),
            pl.BlockSpec((_BM, _DH), lambda i: (i, 0)),
        ],
        out_shape=[
            jax.ShapeDtypeStruct((_N_PAD, _DH), jnp.float32),
            jax.ShapeDtypeStruct((_N_PAD, _DH), jnp.float32),
        ],
    )(o0, o1, deg0, deg1, b, W)


def _final_body(o0_ref, o1_ref, deg0_ref, deg1_ref, b_ref, out_ref):
    dis = _dis_block(deg0_ref, deg1_ref)
    t = jnp.concatenate([o0_ref[...], o1_ref[...]], axis=1)
    out_ref[...] = t * dis + b_ref[...]


def _final_call(o0, o1, deg0, deg1, b):
    return pl.pallas_call(
        _final_body,
        grid=_GRID,
        in_specs=[
            pl.BlockSpec((_BM, _DH), lambda i: (i, 0)),
            pl.BlockSpec((_BM, _DH), lambda i: (i, 0)),
            pl.BlockSpec((_BM, 1), lambda i: (i, 0)),
            pl.BlockSpec((_BM, 1), lambda i: (i, 0)),
            pl.BlockSpec((1, _D), lambda i: (0, 0)),
        ],
        out_specs=pl.BlockSpec((_BM, _D), lambda i: (i, 0)),
        out_shape=jax.ShapeDtypeStruct((_N_PAD, _D), jnp.float32),
    )(o0, o1, deg0, deg1, b)


def kernel(x, edge_index, W1, b1, W2, b2, W3, b3):
    src = edge_index[0].astype(jnp.int32)
    dst = edge_index[1].astype(jnp.int32)
    pad = _EP - _E
    fill = jnp.full((pad,), _N, jnp.int32)
    src_p = jnp.concatenate([src, fill]).reshape(_NCHUNKS, _CH)
    dst_p = jnp.concatenate([dst, fill]).reshape(_NCHUNKS, _CH)
    x_p = jnp.pad(x.astype(jnp.float32), ((0, _N_PAD - _N), (0, 0)))
    b1r = b1.reshape(1, _D)
    b2r = b2.reshape(1, _D)
    b3r = b3.reshape(1, _D)

    deg0, deg1 = _deg_call(dst_p)
    deg0 = deg0[:, :1]
    deg1 = deg1[:, :1]
    hs0, hs1 = _mm1_call(x_p, W1, deg0, deg1)
    o0, o1 = _scatter_call(hs0, hs1, src_p, dst_p)
    hs0, hs1 = _comb_call(o0, o1, deg0, deg1, b1r, W2)
    o0, o1 = _scatter_call(hs0, hs1, src_p, dst_p)
    hs0, hs1 = _comb_call(o0, o1, deg0, deg1, b2r, W3)
    o0, o1 = _scatter_call(hs0, hs1, src_p, dst_p)
    out = _final_call(o0, o1, deg0, deg1, b3r)
    return out[:_N]
